# baseline (speedup 1.0000x reference)
import functools

import jax
import jax.numpy as jnp
from jax import lax
from jax.experimental import pallas as pl
from jax.experimental.pallas import tpu as pltpu
from jax.experimental.pallas import tpu_sc as plsc

NC = 2
NS = 16
NW = NC * NS
LANES = 16
CH = 80


def _proj_body(x_ref, wa_ref, wb_ref, b1_ref, a_ref, b_ref):
    xb = x_ref[...]
    a_ref[...] = (
        jnp.dot(xb, wa_ref[...], preferred_element_type=jnp.float32) + b1_ref[...]
    )
    b_ref[...] = jnp.dot(xb, wb_ref[...], preferred_element_type=jnp.float32)


def _edge_body(per_w, n_chunks, hid,
               a_hbm, b_hbm, row_hbm, col_hbm, prm_hbm, out_hbm,
               idxr_v, idxc_v, arow_v, brow_v, out_v, prm_v, sem_a, sem_b):
    cid = lax.axis_index("c")
    sid = lax.axis_index("s")
    wid = sid * NC + cid
    base = wid * per_w
    pltpu.sync_copy(prm_hbm, prm_v)
    rids0 = lax.iota(jnp.int32, LANES)
    w2v = [prm_v[pl.ds(j * LANES, LANES)] for j in range(hid // LANES)]
    b2s = prm_v[pl.ds(hid, LANES)][0]

    def chunk_body(c, carry):
        cb = base + c * CH
        pltpu.sync_copy(row_hbm.at[pl.ds(cb, CH)], idxr_v)
        pltpu.sync_copy(col_hbm.at[pl.ds(cb, CH)], idxc_v)
        cpa = pltpu.async_copy(a_hbm.at[idxr_v], arow_v, sem_a)
        cpb = pltpu.async_copy(b_hbm.at[idxc_v], brow_v, sem_b)
        cpa.wait()
        cpb.wait()

        def group_body(g, gcarry):
            rids = rids0 + g * LANES
            acc = jnp.zeros((LANES,), jnp.float32)
            for k in range(hid):
                kv = jnp.full((LANES,), k, jnp.int32)
                av = plsc.load_gather(arow_v, [rids, kv])
                bv = plsc.load_gather(brow_v, [rids, kv])
                acc = acc + jnp.maximum(av + bv, 0.0) * w2v[k // LANES][k % LANES]
            z = acc + b2s
            m = 1.0 / (1.0 + jnp.exp(-z))
            out_v[pl.ds(g * LANES, LANES)] = m
            return gcarry

        lax.fori_loop(0, CH // LANES, group_body, 0)
        pltpu.sync_copy(out_v, out_hbm.at[pl.ds(cb, CH)])
        return carry

    lax.fori_loop(0, n_chunks, chunk_body, 0)


def kernel(x, edge_index, W1, b1, W2, b2):
    n, d = x.shape
    hid = W1.shape[0]
    e = edge_index.shape[1]
    row = edge_index[0].astype(jnp.int32)
    col = edge_index[1].astype(jnp.int32)
    wa = W1[:, :d].T
    wb = W1[:, d:].T

    a_tab, b_tab = pl.pallas_call(
        _proj_body,
        out_shape=(
            jax.ShapeDtypeStruct((n, hid), jnp.float32),
            jax.ShapeDtypeStruct((n, hid), jnp.float32),
        ),
    )(x, wa, wb, b1.reshape(1, hid))

    prm = jnp.concatenate(
        [W2.reshape(-1), b2.reshape(-1), jnp.zeros((15,), jnp.float32)]
    )

    per_w = e // NW
    n_chunks = per_w // CH

    edge_fn = pl.kernel(
        functools.partial(_edge_body, per_w, n_chunks, hid),
        out_type=jax.ShapeDtypeStruct((e,), jnp.float32),
        mesh=plsc.VectorSubcoreMesh(core_axis_name="c", subcore_axis_name="s"),
        scratch_types=[
            pltpu.VMEM((CH,), jnp.int32),
            pltpu.VMEM((CH,), jnp.int32),
            pltpu.VMEM((CH, 128), jnp.float32),
            pltpu.VMEM((CH, 128), jnp.float32),
            pltpu.VMEM((CH,), jnp.float32),
            pltpu.VMEM((144,), jnp.float32),
            pltpu.SemaphoreType.DMA,
            pltpu.SemaphoreType.DMA,
        ],
        compiler_params=pltpu.CompilerParams(needs_layout_passes=False),
    )
    return edge_fn(a_tab, b_tab, row, col, prm)


# speedup vs baseline: 1.2248x; 1.2248x over previous
import functools

import jax
import jax.numpy as jnp
from jax import lax
from jax.experimental import pallas as pl
from jax.experimental.pallas import tpu as pltpu
from jax.experimental.pallas import tpu_sc as plsc

NC = 2
NS = 16
NW = NC * NS
LANES = 16
CH = 128
NIDX = 2
NROW = 2
INNER = 2


def _proj_body(x_ref, wa_ref, wb_ref, b1_ref, a_ref, b_ref):
    xb = x_ref[...]
    a_ref[...] = (
        jnp.dot(xb, wa_ref[...], preferred_element_type=jnp.float32) + b1_ref[...]
    )
    b_ref[...] = jnp.dot(xb, wb_ref[...], preferred_element_type=jnp.float32)


def _edge_body(per_w, n_full, tail, hid,
               a_hbm, b_hbm, row_hbm, col_hbm, prm_hbm, out_hbm,
               idxr, idxc, arow, brow, out_v, prm_v, sem_idx, sem_row):
    cid = lax.axis_index("c")
    sid = lax.axis_index("s")
    wid = sid * NC + cid
    base = wid * per_w
    pltpu.sync_copy(prm_hbm, prm_v)
    rids0 = lax.iota(jnp.int32, LANES)
    w2v = [prm_v[pl.ds(j * LANES, LANES)] for j in range(hid // LANES)]
    b2s = prm_v[pl.ds(hid, LANES)][0]

    def start_idx(j, b):
        cb = base + j * CH
        pltpu.async_copy(row_hbm.at[pl.ds(cb, CH)], idxr[b], sem_idx[b])
        pltpu.async_copy(col_hbm.at[pl.ds(cb, CH)], idxc[b], sem_idx[b])

    def wait_idx(j, b):
        cb = base + j * CH
        pltpu.make_async_copy(row_hbm.at[pl.ds(cb, CH)], idxr[b], sem_idx[b]).wait()
        pltpu.make_async_copy(col_hbm.at[pl.ds(cb, CH)], idxc[b], sem_idx[b]).wait()

    def start_rows(bi, br):
        pltpu.async_copy(a_hbm.at[idxr[bi]], arow[br], sem_row[br])
        pltpu.async_copy(b_hbm.at[idxc[bi]], brow[br], sem_row[br])

    def wait_rows(bi, br):
        pltpu.make_async_copy(a_hbm.at[idxr[bi]], arow[br], sem_row[br]).wait()
        pltpu.make_async_copy(b_hbm.at[idxc[bi]], brow[br], sem_row[br]).wait()

    def compute(j, br, n_groups):
        def group_body(g, gcarry):
            rids = rids0 + g * LANES
            acc = jnp.zeros((LANES,), jnp.float32)
            for k in range(hid):
                kv = jnp.full((LANES,), k, jnp.int32)
                av = plsc.load_gather(arow[br], [rids, kv])
                bv = plsc.load_gather(brow[br], [rids, kv])
                acc = acc + jnp.maximum(av + bv, 0.0) * w2v[k // LANES][k % LANES]
            z = acc + b2s
            m = 1.0 / (1.0 + jnp.exp(-z))
            out_v[pl.ds(j * CH + g * LANES, LANES)] = m
            return gcarry

        lax.fori_loop(0, n_groups, group_body, 0)

    start_idx(0, 0)
    start_idx(1, 1)
    wait_idx(0, 0)
    start_rows(0, 0)

    n_outer = (n_full + INNER - 1) // INNER

    def outer(jj, carry):
        for b in range(INNER):
            j = jj * INNER + b
            bi = b % NIDX
            br = b % NROW

            @pl.when(j + 1 < n_full)
            def _():
                wait_idx(j + 1, (bi + 1) % NIDX)
                start_rows((bi + 1) % NIDX, (br + 1) % NROW)

            @pl.when(j < n_full)
            def _():
                wait_rows(bi, br)

            @pl.when(j + NIDX < n_full)
            def _():
                start_idx(j + NIDX, bi)

            @pl.when(j < n_full)
            def _():
                compute(j, br, CH // LANES)
        return carry

    lax.fori_loop(0, n_outer, outer, 0)

    if tail:
        cb = base + n_full * CH
        pltpu.sync_copy(row_hbm.at[pl.ds(cb, tail)], idxr[0].at[pl.ds(0, tail)])
        pltpu.sync_copy(col_hbm.at[pl.ds(cb, tail)], idxc[0].at[pl.ds(0, tail)])
        cpa = pltpu.async_copy(
            a_hbm.at[idxr[0].at[pl.ds(0, tail)]],
            arow[0].at[pl.ds(0, tail), :], sem_row[0])
        cpb = pltpu.async_copy(
            b_hbm.at[idxc[0].at[pl.ds(0, tail)]],
            brow[0].at[pl.ds(0, tail), :], sem_row[0])
        cpa.wait()
        cpb.wait()

        def tail_group(g, gcarry):
            rids = rids0 + g * LANES
            acc = jnp.zeros((LANES,), jnp.float32)
            for k in range(hid):
                kv = jnp.full((LANES,), k, jnp.int32)
                av = plsc.load_gather(arow[0], [rids, kv])
                bv = plsc.load_gather(brow[0], [rids, kv])
                acc = acc + jnp.maximum(av + bv, 0.0) * w2v[k // LANES][k % LANES]
            z = acc + b2s
            m = 1.0 / (1.0 + jnp.exp(-z))
            out_v[pl.ds(n_full * CH + g * LANES, LANES)] = m
            return gcarry

        lax.fori_loop(0, tail // LANES, tail_group, 0)

    pltpu.sync_copy(out_v, out_hbm.at[pl.ds(base, per_w)])


def kernel(x, edge_index, W1, b1, W2, b2):
    n, d = x.shape
    hid = W1.shape[0]
    e = edge_index.shape[1]
    row = edge_index[0].astype(jnp.int32)
    col = edge_index[1].astype(jnp.int32)
    wa = W1[:, :d].T
    wb = W1[:, d:].T

    a_tab, b_tab = pl.pallas_call(
        _proj_body,
        out_shape=(
            jax.ShapeDtypeStruct((n, hid), jnp.float32),
            jax.ShapeDtypeStruct((n, hid), jnp.float32),
        ),
    )(x, wa, wb, b1.reshape(1, hid))

    prm = jnp.concatenate(
        [W2.reshape(-1), b2.reshape(-1), jnp.zeros((15,), jnp.float32)]
    )

    per_w = e // NW
    n_full = per_w // CH
    tail = per_w - n_full * CH

    edge_fn = pl.kernel(
        functools.partial(_edge_body, per_w, n_full, tail, hid),
        out_type=jax.ShapeDtypeStruct((e,), jnp.float32),
        mesh=plsc.VectorSubcoreMesh(core_axis_name="c", subcore_axis_name="s"),
        scratch_types=[
            [pltpu.VMEM((CH,), jnp.int32) for _ in range(NIDX)],
            [pltpu.VMEM((CH,), jnp.int32) for _ in range(NIDX)],
            [pltpu.VMEM((CH, 128), jnp.float32) for _ in range(NROW)],
            [pltpu.VMEM((CH, 128), jnp.float32) for _ in range(NROW)],
            pltpu.VMEM((per_w,), jnp.float32),
            pltpu.VMEM((144,), jnp.float32),
            [pltpu.SemaphoreType.DMA for _ in range(NIDX)],
            [pltpu.SemaphoreType.DMA for _ in range(NROW)],
        ],
        compiler_params=pltpu.CompilerParams(needs_layout_passes=False),
    )
    return edge_fn(a_tab, b_tab, row, col, prm)


# speedup vs baseline: 6.7098x; 5.4785x over previous
import functools

import jax
import jax.numpy as jnp
from jax import lax
from jax.experimental import pallas as pl
from jax.experimental.pallas import tpu as pltpu
from jax.experimental.pallas import tpu_sc as plsc

NC = 2
NS = 16
NW = NC * NS
LANES = 16
CH = 128
NIDX = 4
NROW = 2
INNER = 4


def _proj_body(x_ref, wa_ref, wb_ref, b1_ref, a_ref, b_ref):
    xb = x_ref[...]
    a_ref[...] = (
        jnp.dot(xb, wa_ref[...], preferred_element_type=jnp.float32) + b1_ref[...]
    )
    b_ref[...] = jnp.dot(xb, wb_ref[...], preferred_element_type=jnp.float32)


def _edge_body(per_w, n_full, tail, hid,
               a_hbm, b_hbm, row_hbm, col_hbm, prm_hbm, out_hbm,
               idxr, idxc, arow, brow, out_v, prm_v, tmp_v, sem_idx, sem_row):
    cid = lax.axis_index("c")
    sid = lax.axis_index("s")
    wid = sid * NC + cid
    base = wid * per_w
    pltpu.sync_copy(prm_hbm, prm_v)
    rids0 = lax.iota(jnp.int32, LANES)
    last_lane = rids0 == (LANES - 1)
    w2v = [prm_v[pl.ds(j * LANES, LANES)] for j in range(hid // LANES)]
    b2s = prm_v[pl.ds(hid, LANES)][0]

    def start_idx(j, b):
        cb = base + j * CH
        pltpu.async_copy(row_hbm.at[pl.ds(cb, CH)], idxr[b], sem_idx[b])
        pltpu.async_copy(col_hbm.at[pl.ds(cb, CH)], idxc[b], sem_idx[b])

    def wait_idx(j, b):
        cb = base + j * CH
        pltpu.make_async_copy(row_hbm.at[pl.ds(cb, CH)], idxr[b], sem_idx[b]).wait()
        pltpu.make_async_copy(col_hbm.at[pl.ds(cb, CH)], idxc[b], sem_idx[b]).wait()

    def start_rows(bi, br):
        pltpu.async_copy(a_hbm.at[idxr[bi]], arow[br], sem_row[br])
        pltpu.async_copy(b_hbm.at[idxc[bi]], brow[br], sem_row[br])

    def wait_rows(bi, br):
        pltpu.make_async_copy(a_hbm.at[idxr[bi]], arow[br], sem_row[br]).wait()
        pltpu.make_async_copy(b_hbm.at[idxc[bi]], brow[br], sem_row[br]).wait()

    def compute(j, br, n_groups, tmp_v):
        def group_body(g, gcarry):
            eb = g * LANES
            for e in range(LANES):
                acc = jnp.zeros((LANES,), jnp.float32)
                for jb in range(hid // LANES):
                    av = arow[br][eb + e, pl.ds(jb * LANES, LANES)]
                    bv = brow[br][eb + e, pl.ds(jb * LANES, LANES)]
                    acc = acc + jnp.maximum(av + bv, 0.0) * w2v[jb]
                s = lax.cumsum(acc)
                plsc.store_scatter(
                    tmp_v, [jnp.full((LANES,), e, jnp.int32)], s, mask=last_lane
                )
            z = tmp_v[...] + b2s
            m = 1.0 / (1.0 + jnp.exp(-z))
            out_v[pl.ds(j * CH + g * LANES, LANES)] = m
            return gcarry

        lax.fori_loop(0, n_groups, group_body, 0)

    for b in range(NIDX):
        start_idx(b, b)
    wait_idx(0, 0)
    start_rows(0, 0)

    n_outer = (n_full + INNER - 1) // INNER

    def outer(jj, carry):
        for b in range(INNER):
            j = jj * INNER + b
            bi = b % NIDX
            br = b % NROW

            @pl.when(j + 1 < n_full)
            def _():
                wait_idx(j + 1, (bi + 1) % NIDX)
                start_rows((bi + 1) % NIDX, (br + 1) % NROW)

            @pl.when(j < n_full)
            def _():
                wait_rows(bi, br)

            @pl.when(j + NIDX < n_full)
            def _():
                start_idx(j + NIDX, bi)

            @pl.when(j < n_full)
            def _():
                compute(j, br, CH // LANES, tmp_v)
        return carry

    lax.fori_loop(0, n_outer, outer, 0)

    if tail:
        cb = base + n_full * CH
        pltpu.sync_copy(row_hbm.at[pl.ds(cb, tail)], idxr[0].at[pl.ds(0, tail)])
        pltpu.sync_copy(col_hbm.at[pl.ds(cb, tail)], idxc[0].at[pl.ds(0, tail)])
        cpa = pltpu.async_copy(
            a_hbm.at[idxr[0].at[pl.ds(0, tail)]],
            arow[0].at[pl.ds(0, tail), :], sem_row[0])
        cpb = pltpu.async_copy(
            b_hbm.at[idxc[0].at[pl.ds(0, tail)]],
            brow[0].at[pl.ds(0, tail), :], sem_row[0])
        cpa.wait()
        cpb.wait()

        def tail_group(g, gcarry):
            eb = g * LANES
            for e in range(LANES):
                acc = jnp.zeros((LANES,), jnp.float32)
                for jb in range(hid // LANES):
                    av = arow[0][eb + e, pl.ds(jb * LANES, LANES)]
                    bv = brow[0][eb + e, pl.ds(jb * LANES, LANES)]
                    acc = acc + jnp.maximum(av + bv, 0.0) * w2v[jb]
                s = lax.cumsum(acc)
                plsc.store_scatter(
                    tmp_v, [jnp.full((LANES,), e, jnp.int32)], s, mask=last_lane
                )
            z = tmp_v[...] + b2s
            m = 1.0 / (1.0 + jnp.exp(-z))
            out_v[pl.ds(n_full * CH + g * LANES, LANES)] = m
            return gcarry

        lax.fori_loop(0, tail // LANES, tail_group, 0)

    pltpu.sync_copy(out_v, out_hbm.at[pl.ds(base, per_w)])


def kernel(x, edge_index, W1, b1, W2, b2):
    n, d = x.shape
    hid = W1.shape[0]
    e = edge_index.shape[1]
    row = edge_index[0].astype(jnp.int32)
    col = edge_index[1].astype(jnp.int32)
    wa = W1[:, :d].T
    wb = W1[:, d:].T

    a_tab, b_tab = pl.pallas_call(
        _proj_body,
        out_shape=(
            jax.ShapeDtypeStruct((n, hid), jnp.float32),
            jax.ShapeDtypeStruct((n, hid), jnp.float32),
        ),
    )(x, wa, wb, b1.reshape(1, hid))

    prm = jnp.concatenate(
        [W2.reshape(-1), b2.reshape(-1), jnp.zeros((15,), jnp.float32)]
    )

    per_w = e // NW
    n_full = per_w // CH
    tail = per_w - n_full * CH

    edge_fn = pl.kernel(
        functools.partial(_edge_body, per_w, n_full, tail, hid),
        out_type=jax.ShapeDtypeStruct((e,), jnp.float32),
        mesh=plsc.VectorSubcoreMesh(core_axis_name="c", subcore_axis_name="s"),
        scratch_types=[
            [pltpu.VMEM((CH,), jnp.int32) for _ in range(NIDX)],
            [pltpu.VMEM((CH,), jnp.int32) for _ in range(NIDX)],
            [pltpu.VMEM((CH, 128), jnp.float32) for _ in range(NROW)],
            [pltpu.VMEM((CH, 128), jnp.float32) for _ in range(NROW)],
            pltpu.VMEM((per_w,), jnp.float32),
            pltpu.VMEM((144,), jnp.float32),
            pltpu.VMEM((LANES,), jnp.float32),
            [pltpu.SemaphoreType.DMA for _ in range(NIDX)],
            [pltpu.SemaphoreType.DMA for _ in range(NROW)],
        ],
        compiler_params=pltpu.CompilerParams(needs_layout_passes=False),
    )
    return edge_fn(a_tab, b_tab, row, col, prm)
